# Initial kernel scaffold; baseline (speedup 1.0000x reference)
import jax
import jax.numpy as jnp
from jax.experimental import pallas as pl


def kernel(x, pos, batch, params):
    raise NotImplementedError("write your pallas kernel here")


# speedup vs baseline: 1.2013x; 1.2013x over previous
import numpy as np
import jax
import jax.numpy as jnp
from jax.experimental import pallas as pl

N_NODES = 10000
HIDDEN = 128
OUT_DIM = 128
K_NN = 6
ALPHA = 0.2
RATIOS = (0.15, 0.25, 0.5)

_BIG = 1.0e30


def _knn_body(a_ref, at_ref, o_ref, *, n, npad, k):
    p = a_ref[...]
    qt = at_ref[...]
    pb = p.astype(jnp.bfloat16).astype(jnp.float32)
    qb = qt.astype(jnp.bfloat16).astype(jnp.float32)
    acc = pb[:, 0:1] * qb[0:1, :]
    acc = acc + pb[:, 1:2] * qb[1:2, :]
    acc = acc + pb[:, 2:3] * qb[2:3, :]
    sqr = p[:, 0:1] * p[:, 0:1] + p[:, 1:2] * p[:, 1:2] + p[:, 2:3] * p[:, 2:3]
    sqc = qt[0:1, :] * qt[0:1, :] + qt[1:2, :] * qt[1:2, :] + qt[2:3, :] * qt[2:3, :]
    d2 = sqr + sqc - 2.0 * acc
    R = p.shape[0]
    colid = jax.lax.broadcasted_iota(jnp.int32, (R, npad), 1)
    d2 = jnp.where(colid >= n, _BIG, d2)
    outs = []
    for _ in range(k):
        m = jnp.min(d2, axis=1, keepdims=True)
        sel = jnp.min(jnp.where(d2 <= m, colid, n), axis=1, keepdims=True)
        outs.append(sel)
        d2 = jnp.where(colid == sel, _BIG, d2)
    outs.append(jnp.zeros((R, 128 - k), jnp.int32))
    o_ref[...] = jnp.concatenate(outs, axis=1)


def _knn_graph(pos, k):
    n = pos.shape[0]
    npad = max(128, -(-n // 128) * 128)
    R = 128
    a = jnp.zeros((npad, 128), jnp.float32).at[:n, :3].set(pos)
    at = jnp.zeros((8, npad), jnp.float32).at[:3, :n].set(pos.T)
    import functools
    body = functools.partial(_knn_body, n=n, npad=npad, k=k)
    out = pl.pallas_call(
        body,
        grid=(npad // R,),
        in_specs=[
            pl.BlockSpec((R, 128), lambda i: (i, 0)),
            pl.BlockSpec((8, npad), lambda i: (0, 0)),
        ],
        out_specs=pl.BlockSpec((R, 128), lambda i: (i, 0)),
        out_shape=jax.ShapeDtypeStruct((npad, 128), jnp.int32),
    )(a, at)
    nbr = out[:n, :k]
    centers = jnp.asarray(np.repeat(np.arange(n, dtype=np.int32), k))
    return jnp.stack([nbr.reshape(-1).astype(jnp.int32), centers], axis=0)


def _to_undirected(edge_index, N):
    r = jnp.concatenate([edge_index[0], edge_index[1]]).astype(jnp.int32)
    c = jnp.concatenate([edge_index[1], edge_index[0]]).astype(jnp.int32)
    code = jnp.sort(r * N + c)
    mask = jnp.concatenate([jnp.ones((1,), jnp.float32),
                            (code[1:] != code[:-1]).astype(jnp.float32)])
    return jnp.stack([code // N, code % N], axis=0), mask


def _edge_weights(p, pos, ei, mask):
    row = ei[0]; col = ei[1]
    d = jnp.linalg.norm(pos[row] - pos[col] + 0.0, axis=1)[:, None]
    h = d @ p['w1'] + p['b1']
    cnt = jnp.sum(mask)
    mu = jnp.sum(h * mask[:, None], axis=0) / cnt
    var = jnp.sum(mask[:, None] * (h - mu) ** 2, axis=0) / cnt
    h = (h - mu) / jnp.sqrt(var + 1e-5) * p['gamma'] + p['beta']
    h = jax.nn.relu(h)
    w = (h @ p['w2'] + p['b2']).reshape(-1)
    return jax.nn.relu(w) * mask


def _gcn2(W, x, x0, ei, ew):
    N = x.shape[0]
    row = ei[0]; col = ei[1]
    deg = jax.ops.segment_sum(ew, col, num_segments=N)
    dinv = jnp.where(deg > 0, jax.lax.rsqrt(jnp.maximum(deg, 1e-12)), 0.0)
    norm = dinv[row] * ew * dinv[col]
    agg = jax.ops.segment_sum(norm[:, None] * x[row], col, num_segments=N)
    h = (1.0 - ALPHA) * agg + ALPHA * x0
    return h @ W


def _pool(p, x, ei, ew, mask, ratio):
    N = x.shape[0]
    row = ei[0]; col = ei[1]
    xp = jax.ops.segment_sum(ew[:, None] * x[row], col, num_segments=N) @ p['gnn_wrel'] + p['gnn_brel'] + x @ p['gnn_wroot']
    xpj = xp[row]
    xq = jax.ops.segment_max(xpj, col, num_segments=N)
    xq = (xq @ p['lin_w'] + p['lin_b'])[col]
    s = (jnp.concatenate([xq, xpj], axis=1) @ p['att_w'] + p['att_b']).reshape(-1)
    s = jax.nn.leaky_relu(s, 0.2)
    m = jax.ops.segment_max(s, col, num_segments=N)
    e = jnp.exp(s - m[col]) * mask
    den = jax.ops.segment_sum(e, col, num_segments=N)
    s = e / (den[col] + 1e-16)
    xnew = jax.ops.segment_sum(s[:, None] * x[row], col, num_segments=N)
    a = xnew @ p['le_w1'] + p['le_b1']
    b = xnew @ p['le_w2']
    msg = ew[:, None] * (a[row] - b[col])
    fit = jax.nn.sigmoid((jax.ops.segment_sum(msg, col, num_segments=N) + xnew @ p['le_w3'] + p['le_b3']).reshape(-1))
    kk = int(np.ceil(ratio * N))
    perm = jnp.argsort(-fit)[:kk]
    xout = xnew[perm] * fit[perm][:, None]
    return xout, perm


def kernel(x, pos, batch, params):
    ei, mask = _to_undirected(_knn_graph(pos, K_NN), pos.shape[0])
    ew = _edge_weights(params['edge_mlp0'], pos, ei, mask)
    h = jax.nn.relu(_gcn2(params['conv1_w'], x, x, ei, ew))
    h, perm = _pool(params['pool1'], h, ei, ew, mask, RATIOS[0])
    x1 = x[perm]; pos1 = pos[perm]
    ei, mask = _to_undirected(_knn_graph(pos1, K_NN), pos1.shape[0])
    ew = _edge_weights(params['edge_mlp1'], pos1, ei, mask)
    readout1 = jnp.concatenate([jnp.mean(h, axis=0, keepdims=True), jnp.max(h, axis=0, keepdims=True)], axis=1)
    h = jax.nn.relu(_gcn2(params['conv2_w'], h, x1, ei, ew))
    h, perm = _pool(params['pool2'], h, ei, ew, mask, RATIOS[1])
    x2 = x1[perm]; pos2 = pos1[perm]
    ei, mask = _to_undirected(_knn_graph(pos2, K_NN), pos2.shape[0])
    ew = _edge_weights(params['edge_mlp2'], pos2, ei, mask)
    readout2 = jnp.concatenate([jnp.mean(h, axis=0, keepdims=True), jnp.max(h, axis=0, keepdims=True)], axis=1)
    h = jax.nn.relu(_gcn2(params['conv3_w'], h, x2, ei, ew))
    h, perm = _pool(params['pool3'], h, ei, ew, mask, RATIOS[2])
    x3 = x2[perm]; pos3 = pos2[perm]
    ei, mask = _to_undirected(_knn_graph(pos3, K_NN), pos3.shape[0])
    ew = _edge_weights(params['edge_mlp3'], pos3, ei, mask)
    h = jax.nn.relu(_gcn2(params['conv4_w'], h, x3, ei, ew))
    gate = jax.nn.softmax(h @ params['gate_w'] + params['gate_b'], axis=0)
    pooled = jnp.sum(gate * (h @ params['nn_w'] + params['nn_b']), axis=0, keepdims=True)
    out = jnp.concatenate([pooled, readout2, readout1], axis=1)
    return out
